# baseline (speedup 1.0000x reference)
import jax
import jax.numpy as jnp
from jax import lax
from jax.experimental import pallas as pl
from jax.experimental.pallas import tpu as pltpu
from jax.experimental.pallas import tpu_sc as plsc

NUM_NTYPE = 4
TBL = 250000
EMBED = 64
B = 16384

NC = 2
NS = 16
NW = NC * NS
L = 16

GRP = 8
ROWS_PER_TILE = B // NW
NGROUP = ROWS_PER_TILE // L


def _body(tids_hbm, xids_hbm, e0, e1, e2, e3, out_hbm,
          tids_v, xids_v, slots, semA, semB, semOA, semOB):
    embs = (e0, e1, e2, e3)
    wid = lax.axis_index("s") * NC + lax.axis_index("c")
    base = wid * ROWS_PER_TILE

    pltpu.sync_copy(tids_hbm.at[pl.ds(base, ROWS_PER_TILE)], tids_v)
    pltpu.sync_copy(xids_hbm.at[pl.ds(base, ROWS_PER_TILE)], xids_v)

    def issue(g, bank, sem):
        tv = tids_v[pl.ds(g * L, L)]
        xv = xids_v[pl.ds(g * L, L)]
        gv = xv >> 3
        sv = xv & jnp.int32(GRP - 1)
        for k in range(L):
            t = tv[k]
            g3 = gv[k]
            sub = sv[k]
            dst = slots.at[pl.ds(bank * L + k, 1)]
            for tt in range(NUM_NTYPE):
                @pl.when(t == jnp.int32(tt))
                def _(tt=tt, g3=g3, sub=sub, dst=dst):
                    pltpu.async_copy(
                        embs[tt].at[pl.ds(g3, 1), pl.ds(sub, 1)], dst, sem)

    def drain(bank, sem):
        pltpu.make_async_copy(
            e0.at[pl.ds(0, L), pl.ds(0, 1)],
            slots.at[pl.ds(bank * L, L)], sem).wait()

    def out(g, bank, semo):
        for k in range(L):
            q = (base + g * L + k) // GRP
            pltpu.async_copy(
                slots.at[pl.ds(bank * L + k, 1)],
                out_hbm.at[pl.ds(q, 1), pl.ds(k % GRP, 1)], semo)

    issue(0, 0, semA)
    issue(1, 1, semB)

    def step(i, _):
        drain(0, semA)
        out(2 * i, 0, semOA)

        @pl.when(i < NGROUP // 2 - 1)
        def _():
            drain(0, semOA)
            issue(2 * i + 2, 0, semA)

        drain(1, semB)
        out(2 * i + 1, 1, semOB)

        @pl.when(i < NGROUP // 2 - 1)
        def _():
            drain(1, semOB)
            issue(2 * i + 3, 1, semB)
        return 0
    lax.fori_loop(0, NGROUP // 2, step, 0)

    drain(0, semOA)
    drain(1, semOB)


@jax.jit
def _run(node_tids, type_ids, emb0, emb1, emb2, emb3):
    mesh = plsc.VectorSubcoreMesh(
        core_axis_name="c", subcore_axis_name="s",
        num_cores=NC, num_subcores=NS)
    grouped = [e.reshape(TBL // GRP, GRP, EMBED)
               for e in (emb0, emb1, emb2, emb3)]
    out3 = pl.kernel(
        _body,
        out_type=jax.ShapeDtypeStruct((B // GRP, GRP, EMBED), jnp.float32),
        mesh=mesh,
        scratch_types=[
            pltpu.VMEM((ROWS_PER_TILE,), jnp.int32),
            pltpu.VMEM((ROWS_PER_TILE,), jnp.int32),
            pltpu.VMEM((2 * L, 1, EMBED), jnp.float32),
            pltpu.SemaphoreType.DMA,
            pltpu.SemaphoreType.DMA,
            pltpu.SemaphoreType.DMA,
            pltpu.SemaphoreType.DMA,
        ],
    )(node_tids, type_ids, *grouped)
    return out3.reshape(B, EMBED)


def kernel(node_ids, node_tids, type_ids, emb0, emb1, emb2, emb3):
    del node_ids
    return _run(node_tids.astype(jnp.int32), type_ids.astype(jnp.int32),
                emb0, emb1, emb2, emb3)
